# baseline (speedup 1.0000x reference)
import functools

import jax
import jax.numpy as jnp
from jax import lax
from jax.experimental import pallas as pl
from jax.experimental.pallas import tpu as pltpu
from jax.experimental.pallas import tpu_sc as plsc

_B, _S, _D, _V = 4, 2048, 1024, 100000
_NC, _NS = 2, 16
_NW = _NC * _NS
_RPW = (_B * _S) // _NW
_C = 32
_NCH = _RPW // _C


def _emb_body(idx_hbm, tok_hbm, wpe_hbm, out_hbm, idxc, tok_v, wpe_v, sem):
    wid = lax.axis_index("s") * _NC + lax.axis_index("c")
    base = wid * _RPW
    s0 = lax.rem(base, _S)
    for c in range(_NCH):
        off = c * _C
        pltpu.sync_copy(idx_hbm.at[pl.ds(base + off, _C)], idxc)
        gcp = pltpu.async_copy(tok_hbm.at[idxc], tok_v, sem)
        pltpu.sync_copy(wpe_hbm.at[pl.ds(s0 + off, _C)], wpe_v)
        gcp.wait()

        def _add_row(r, carry):
            for j in range(_D // 16):
                sl = pl.ds(j * 16, 16)
                tok_v[r, sl] = tok_v[r, sl] + wpe_v[r, sl]
            return carry

        lax.fori_loop(0, _C, _add_row, 0)
        pltpu.sync_copy(tok_v, out_hbm.at[pl.ds(base + off, _C)])


_sc_embed = functools.partial(
    pl.kernel,
    out_type=jax.ShapeDtypeStruct((_B * _S, _D), jnp.float32),
    mesh=plsc.VectorSubcoreMesh(core_axis_name="c", subcore_axis_name="s"),
    scratch_types=[
        pltpu.VMEM((_C,), jnp.int32),
        pltpu.VMEM((_C, _D), jnp.float32),
        pltpu.VMEM((_C, _D), jnp.float32),
        pltpu.SemaphoreType.DMA,
    ],
)(_emb_body)


def kernel(idx, tok_emb, wpe):
    flat = _sc_embed(idx.reshape(_B * _S), tok_emb, wpe)
    return flat.reshape(_B, _S, _D)


# speedup vs baseline: 1.3273x; 1.3273x over previous
import functools

import jax
import jax.numpy as jnp
from jax import lax
from jax.experimental import pallas as pl
from jax.experimental.pallas import tpu as pltpu
from jax.experimental.pallas import tpu_sc as plsc

_B, _S, _D, _V = 4, 2048, 1024, 100000
_NC, _NS = 2, 16
_NW = _NC * _NS
_RPW = (_B * _S) // _NW
_C = 16
_NCH = _RPW // _C


def _emb_body(idx_hbm, tok_hbm, wpe_hbm, out_hbm, idx_v,
              tok0, tok1, wpe0, wpe1,
              gsem0, gsem1, wsem0, wsem1, osem0, osem1):
    toks = [tok0, tok1]
    wpes = [wpe0, wpe1]
    gsems = [gsem0, gsem1]
    wsems = [wsem0, wsem1]
    osems = [osem0, osem1]

    wid = lax.axis_index("s") * _NC + lax.axis_index("c")
    base = wid * _RPW
    s0 = lax.rem(base, _S)
    pltpu.sync_copy(idx_hbm.at[pl.ds(base, _RPW)], idx_v)

    gcp = [None, None]
    wcp = [None, None]
    ocp = [None, None]

    def issue(c):
        p = c % 2
        gcp[p] = pltpu.async_copy(
            tok_hbm.at[idx_v.at[pl.ds(c * _C, _C)]], toks[p], gsems[p])
        wcp[p] = pltpu.async_copy(
            wpe_hbm.at[pl.ds(s0 + c * _C, _C)], wpes[p], wsems[p])

    def finish(c):
        p = c % 2
        gcp[p].wait()
        wcp[p].wait()

        def _add_row(r, carry):
            for j in range(_D // 16):
                sl = pl.ds(j * 16, 16)
                toks[p][r, sl] = toks[p][r, sl] + wpes[p][r, sl]
            return carry

        lax.fori_loop(0, _C, _add_row, 0)
        ocp[p] = pltpu.async_copy(
            toks[p], out_hbm.at[pl.ds(base + c * _C, _C)], osems[p])

    issue(0)
    for c in range(1, _NCH):
        p = c % 2
        if ocp[p] is not None:
            ocp[p].wait()
        issue(c)
        finish(c - 1)
    finish(_NCH - 1)
    ocp[0].wait()
    ocp[1].wait()


_sc_embed = functools.partial(
    pl.kernel,
    out_type=jax.ShapeDtypeStruct((_B * _S, _D), jnp.float32),
    mesh=plsc.VectorSubcoreMesh(core_axis_name="c", subcore_axis_name="s"),
    scratch_types=[
        pltpu.VMEM((_RPW,), jnp.int32),
        pltpu.VMEM((_C, _D), jnp.float32),
        pltpu.VMEM((_C, _D), jnp.float32),
        pltpu.VMEM((_C, _D), jnp.float32),
        pltpu.VMEM((_C, _D), jnp.float32),
        pltpu.SemaphoreType.DMA,
        pltpu.SemaphoreType.DMA,
        pltpu.SemaphoreType.DMA,
        pltpu.SemaphoreType.DMA,
        pltpu.SemaphoreType.DMA,
        pltpu.SemaphoreType.DMA,
    ],
)(_emb_body)


def kernel(idx, tok_emb, wpe):
    flat = _sc_embed(idx.reshape(_B * _S), tok_emb, wpe)
    return flat.reshape(_B, _S, _D)
